# baseline (speedup 1.0000x reference)
import functools

import jax
import jax.numpy as jnp
from jax import lax
from jax.experimental import pallas as pl
from jax.experimental.pallas import tpu as pltpu
from jax.experimental.pallas import tpu_sc as plsc

N = 10000
N_PAD = 10240
E = 160000
H = 256
HH = 128
NC = 2
NS = 16
CHUNK = 125

DEG_CHUNKS = E // (NC * NS * CHUNK)

NH = 5120
ACHUNK = 112
AGG_CHUNKS = 98
E_TILE = AGG_CHUNKS * ACHUNK
E_PAD = NS * E_TILE
PAD_DST = 1 << 20

_mesh = plsc.VectorSubcoreMesh(core_axis_name="c", subcore_axis_name="s")


def _fill_const_1d(ref, n16, value):
    v = jnp.full((16,), value, dtype=jnp.float32)
    for k in range(n16):
        ref[pl.ds(k * 16, 16)] = v


@functools.partial(
    pl.kernel,
    out_type=[jax.ShapeDtypeStruct((N_PAD,), jnp.float32),
              jax.ShapeDtypeStruct((N_PAD,), jnp.float32)],
    mesh=_mesh,
    scratch_types=[
        pltpu.VMEM((DEG_CHUNKS, CHUNK), jnp.int32),
        pltpu.VMEM((128,), jnp.float32),
        pltpu.VMEM((640,), jnp.float32),
        pltpu.VMEM_SHARED((N_PAD,), jnp.float32),
    ],
)
def _sc_degrees(dst_hbm, deg0_hbm, deg1_hbm, idx_v, ones_v, zline_v, acc_sh):
    c = lax.axis_index("c")
    s = lax.axis_index("s")
    wid = c * NS + s
    pltpu.sync_copy(dst_hbm.at[wid], idx_v)
    _fill_const_1d(ones_v, 8, 1.0)
    _fill_const_1d(zline_v, 40, 0.0)
    pltpu.sync_copy(zline_v, acc_sh.at[pl.ds(s * 640, 640)])
    plsc.subcore_barrier()

    def body(j, carry):
        pltpu.sync_copy(ones_v.at[pl.ds(0, CHUNK)], acc_sh.at[idx_v.at[j]],
                        add=True)
        return carry

    lax.fori_loop(0, DEG_CHUNKS, body, 0)
    plsc.subcore_barrier()

    @pl.when(c == 0)
    def _():
        pltpu.sync_copy(acc_sh.at[pl.ds(s * 640, 640)],
                        deg0_hbm.at[pl.ds(s * 640, 640)])

    @pl.when(c == 1)
    def _():
        pltpu.sync_copy(acc_sh.at[pl.ds(s * 640, 640)],
                        deg1_hbm.at[pl.ds(s * 640, 640)])


@functools.partial(
    pl.kernel,
    out_type=jax.ShapeDtypeStruct((NC, N_PAD, HH), jnp.float32),
    mesh=_mesh,
    scratch_types=[
        pltpu.VMEM((AGG_CHUNKS, ACHUNK), jnp.int32),
        pltpu.VMEM((AGG_CHUNKS, ACHUNK), jnp.int32),
        pltpu.VMEM((AGG_CHUNKS, ACHUNK), jnp.int32),
        pltpu.VMEM((ACHUNK, HH), jnp.float32),
        pltpu.VMEM((ACHUNK, HH), jnp.float32),
        pltpu.VMEM((128, HH), jnp.float32),
        pltpu.VMEM_SHARED((NH + 8, HH), jnp.float32),
        pltpu.SemaphoreType.DMA,
        pltpu.SemaphoreType.DMA,
    ],
)
def _sc_aggregate(y_hbm, src_hbm, dst_hbm, agg_hbm,
                  src_v, dst_v, dstp_v, buf0, buf1, zbuf, acc_sh, sem0, sem1):
    c = lax.axis_index("c")
    s = lax.axis_index("s")
    tbl = y_hbm.at[c]
    pltpu.sync_copy(src_hbm.at[s], src_v)
    pltpu.sync_copy(dst_hbm.at[s], dst_v)

    def zrow(r, carry):
        z = jnp.zeros((16,), jnp.float32)
        for k in range(HH // 16):
            zbuf[r, pl.ds(k * 16, 16)] = z
        return carry

    lax.fori_loop(0, 128, zrow, 0)

    for p in range(2):
        base = p * NH

        def remap(j, carry):
            for k in range(ACHUNK // 16):
                d = dst_v[j, pl.ds(k * 16, 16)]
                inr = (d >= base) & (d < base + NH)
                dstp_v[j, pl.ds(k * 16, 16)] = jnp.where(inr, d - base, NH)
            return carry

        lax.fori_loop(0, AGG_CHUNKS, remap, 0)

        zb = s * 320
        pltpu.sync_copy(zbuf, acc_sh.at[pl.ds(zb, 128)])
        pltpu.sync_copy(zbuf, acc_sh.at[pl.ds(zb + 128, 128)])
        pltpu.sync_copy(zbuf.at[pl.ds(0, 64)], acc_sh.at[pl.ds(zb + 256, 64)])

        @pl.when(s == 0)
        def _():
            pltpu.sync_copy(zbuf.at[pl.ds(0, 8)], acc_sh.at[pl.ds(NH, 8)])

        plsc.subcore_barrier()

        pltpu.async_copy(tbl.at[src_v.at[0]], buf0, sem0)
        pltpu.async_copy(tbl.at[src_v.at[1]], buf1, sem1)

        def body(jj, carry):
            j0 = jj * 2
            pltpu.make_async_copy(tbl.at[src_v.at[j0]], buf0, sem0).wait()
            pltpu.sync_copy(buf0, acc_sh.at[dstp_v.at[j0]], add=True)

            @pl.when(jj < AGG_CHUNKS // 2 - 1)
            def _():
                pltpu.async_copy(tbl.at[src_v.at[j0 + 2]], buf0, sem0)

            pltpu.make_async_copy(tbl.at[src_v.at[j0 + 1]], buf1, sem1).wait()
            pltpu.sync_copy(buf1, acc_sh.at[dstp_v.at[j0 + 1]], add=True)

            @pl.when(jj < AGG_CHUNKS // 2 - 1)
            def _():
                pltpu.async_copy(tbl.at[src_v.at[j0 + 3]], buf1, sem1)

            return carry

        lax.fori_loop(0, AGG_CHUNKS // 2, body, 0)
        plsc.subcore_barrier()
        pltpu.sync_copy(acc_sh.at[pl.ds(s * 320, 320)],
                        agg_hbm.at[c, pl.ds(base + s * 320, 320)])


def _tc_dense_body(x_ref, th_ref, wt_ref, bt_ref, wh_ref, deg_ref,
                   y2_ref, gate_ref, gh_ref):
    xb = x_ref[...]
    d = deg_ref[:, 0] + deg_ref[:, 1]
    norm = lax.rsqrt(d + 1e-6)[:, None]
    y = jnp.dot(xb, th_ref[...], preferred_element_type=jnp.float32) * norm
    y2_ref[0] = y[:, :HH]
    y2_ref[1] = y[:, HH:]
    gate = jax.nn.sigmoid(
        jnp.dot(xb, wt_ref[...], preferred_element_type=jnp.float32)
        + bt_ref[0])
    gate_ref[...] = gate
    gh_ref[...] = (1.0 - gate) * jnp.dot(
        xb, wh_ref[...], preferred_element_type=jnp.float32)


_R = 1000


def _tc_dense(x, theta, W_t, b_t, W_h, degT):
    grid = (N // _R,)
    return pl.pallas_call(
        _tc_dense_body,
        grid=grid,
        in_specs=[
            pl.BlockSpec((_R, H), lambda i: (i, 0)),
            pl.BlockSpec((H, H), lambda i: (0, 0)),
            pl.BlockSpec((H, H), lambda i: (0, 0)),
            pl.BlockSpec((1, H), lambda i: (0, 0)),
            pl.BlockSpec((H, H), lambda i: (0, 0)),
            pl.BlockSpec((_R, NC), lambda i: (i, 0)),
        ],
        out_specs=[
            pl.BlockSpec((NC, _R, HH), lambda i: (0, i, 0)),
            pl.BlockSpec((_R, H), lambda i: (i, 0)),
            pl.BlockSpec((_R, H), lambda i: (i, 0)),
        ],
        out_shape=[
            jax.ShapeDtypeStruct((NC, N, HH), jnp.float32),
            jax.ShapeDtypeStruct((N, H), jnp.float32),
            jax.ShapeDtypeStruct((N, H), jnp.float32),
        ],
    )(x, theta, W_t, b_t, W_h, degT)


def _tc_final_body(agg_ref, gate_ref, gh_ref, deg_ref, out_ref):
    agg = jnp.concatenate([agg_ref[0], agg_ref[1]], axis=1)
    d = deg_ref[:, 0] + deg_ref[:, 1]
    norm = lax.rsqrt(d + 1e-6)[:, None]
    gate = gate_ref[...]
    z = gate * (agg * norm) + gh_ref[...]
    out_ref[...] = jnp.where(z > 0, z, jnp.exp(jnp.minimum(z, 0.0)) - 1.0)


def _tc_final(agg2, gate, gh, degT):
    grid = (N // _R,)
    return pl.pallas_call(
        _tc_final_body,
        grid=grid,
        in_specs=[
            pl.BlockSpec((NC, _R, HH), lambda i: (0, i, 0)),
            pl.BlockSpec((_R, H), lambda i: (i, 0)),
            pl.BlockSpec((_R, H), lambda i: (i, 0)),
            pl.BlockSpec((_R, NC), lambda i: (i, 0)),
        ],
        out_specs=pl.BlockSpec((_R, H), lambda i: (i, 0)),
        out_shape=jax.ShapeDtypeStruct((N, H), jnp.float32),
    )(agg2, gate, gh, degT)


def kernel(x, edge_index, W_t, b_t, W_h, theta):
    ei = edge_index.astype(jnp.int32)
    dst = ei[0]
    src = ei[1]
    dst_d = dst.reshape(NC * NS, DEG_CHUNKS, CHUNK)
    deg0, deg1 = _sc_degrees(dst_d)
    degT = jnp.stack([deg0[:N], deg1[:N]], axis=1)
    y2, gate, gh = _tc_dense(x, theta, W_t, b_t.reshape(1, H), W_h, degT)
    pad = E_PAD - E
    src_r = jnp.concatenate(
        [src, jnp.zeros((pad,), jnp.int32)]).reshape(NS, AGG_CHUNKS, ACHUNK)
    dst_r = jnp.concatenate(
        [dst, jnp.full((pad,), PAD_DST, jnp.int32)]).reshape(
            NS, AGG_CHUNKS, ACHUNK)
    agg2 = _sc_aggregate(y2, src_r, dst_r)
    return _tc_final(agg2, gate, gh, degT)


# speedup vs baseline: 2.4552x; 2.4552x over previous
import functools

import jax
import jax.numpy as jnp
from jax import lax
from jax.experimental import pallas as pl
from jax.experimental.pallas import tpu as pltpu
from jax.experimental.pallas import tpu_sc as plsc

N = 10000
N_PAD = 10240
E = 160000
H = 256
HH = 128
NC = 2
NS = 16
CHUNK = 125

DEG_CHUNKS = E // (NC * NS * CHUNK)

ACHUNK = 128
AGG_CHUNKS = 84
E_TILE = AGG_CHUNKS * ACHUNK
E_PAD = NS * E_TILE
_IDEPTH = 4
_NBUF = 2

_mesh = plsc.VectorSubcoreMesh(core_axis_name="c", subcore_axis_name="s")


def _fill_const_1d(ref, n16, value):
    v = jnp.full((16,), value, dtype=jnp.float32)
    for k in range(n16):
        ref[pl.ds(k * 16, 16)] = v


@functools.partial(
    pl.kernel,
    out_type=[jax.ShapeDtypeStruct((N_PAD,), jnp.float32),
              jax.ShapeDtypeStruct((N_PAD,), jnp.float32)],
    mesh=_mesh,
    scratch_types=[
        pltpu.VMEM((DEG_CHUNKS, CHUNK), jnp.int32),
        pltpu.VMEM((128,), jnp.float32),
        pltpu.VMEM((640,), jnp.float32),
        pltpu.VMEM_SHARED((N_PAD,), jnp.float32),
    ],
)
def _sc_degrees(dst_hbm, deg0_hbm, deg1_hbm, idx_v, ones_v, zline_v, acc_sh):
    c = lax.axis_index("c")
    s = lax.axis_index("s")
    wid = c * NS + s
    pltpu.sync_copy(dst_hbm.at[wid], idx_v)
    _fill_const_1d(ones_v, 8, 1.0)
    _fill_const_1d(zline_v, 40, 0.0)
    pltpu.sync_copy(zline_v, acc_sh.at[pl.ds(s * 640, 640)])
    plsc.subcore_barrier()

    def body(j, carry):
        pltpu.sync_copy(ones_v.at[pl.ds(0, CHUNK)], acc_sh.at[idx_v.at[j]],
                        add=True)
        return carry

    lax.fori_loop(0, DEG_CHUNKS, body, 0)
    plsc.subcore_barrier()

    @pl.when(c == 0)
    def _():
        pltpu.sync_copy(acc_sh.at[pl.ds(s * 640, 640)],
                        deg0_hbm.at[pl.ds(s * 640, 640)])

    @pl.when(c == 1)
    def _():
        pltpu.sync_copy(acc_sh.at[pl.ds(s * 640, 640)],
                        deg1_hbm.at[pl.ds(s * 640, 640)])


@functools.partial(
    pl.kernel,
    out_type=jax.ShapeDtypeStruct((NC, N_PAD, HH), jnp.float32),
    mesh=_mesh,
    scratch_types=[
        pltpu.VMEM((_IDEPTH, ACHUNK), jnp.int32),
        pltpu.VMEM((_IDEPTH, ACHUNK), jnp.int32),
    ] + [pltpu.VMEM((ACHUNK, HH), jnp.float32) for _ in range(_NBUF)]
    + [pltpu.VMEM_SHARED((N_PAD, HH), jnp.float32)]
    + [pltpu.SemaphoreType.DMA for _ in range(2 * _IDEPTH + _NBUF)],
)
def _sc_aggregate(y_hbm, src_hbm, dst_hbm, agg_hbm, srci, dsti, *rest):
    bufs = rest[:_NBUF]
    acc_sh = rest[_NBUF]
    sem_si = rest[_NBUF + 1:_NBUF + 1 + _IDEPTH]
    sem_di = rest[_NBUF + 1 + _IDEPTH:_NBUF + 1 + 2 * _IDEPTH]
    sem_g = rest[_NBUF + 1 + 2 * _IDEPTH:]
    c = lax.axis_index("c")
    s = lax.axis_index("s")
    tbl = y_hbm.at[c]

    def idx_fetch(j, q):
        pltpu.async_copy(src_hbm.at[s, j], srci.at[q], sem_si[q])
        pltpu.async_copy(dst_hbm.at[s, j], dsti.at[q], sem_di[q])

    for q in range(_IDEPTH):
        idx_fetch(q, q)

    def zrow(r, carry):
        z = jnp.zeros((16,), jnp.float32)
        for k in range(HH // 16):
            bufs[0][r, pl.ds(k * 16, 16)] = z
        return carry

    lax.fori_loop(0, ACHUNK, zrow, 0)
    for t in range(5):
        pltpu.sync_copy(bufs[0], acc_sh.at[pl.ds(s * 640 + t * 128, 128)])
    plsc.subcore_barrier()

    for b in range(_NBUF):
        pltpu.make_async_copy(src_hbm.at[s, b], srci.at[b], sem_si[b]).wait()
        pltpu.async_copy(tbl.at[srci.at[b]], bufs[b], sem_g[b])

    def body(jj, carry):
        j0 = jj * _IDEPTH
        for u in range(_IDEPTH):
            b = u % _NBUF
            pltpu.make_async_copy(tbl.at[srci.at[u]], bufs[b],
                                  sem_g[b]).wait()
            pltpu.make_async_copy(dst_hbm.at[s, j0 + u], dsti.at[u],
                                  sem_di[u]).wait()
            pltpu.sync_copy(bufs[b], acc_sh.at[dsti.at[u]], add=True)

            @pl.when(j0 + u + _IDEPTH < AGG_CHUNKS)
            def _():
                idx_fetch(j0 + u + _IDEPTH, u)

            @pl.when(j0 + u + _NBUF < AGG_CHUNKS)
            def _():
                q = (u + _NBUF) % _IDEPTH
                pltpu.make_async_copy(src_hbm.at[s, j0 + u + _NBUF],
                                      srci.at[q], sem_si[q]).wait()
                pltpu.async_copy(tbl.at[srci.at[q]], bufs[b], sem_g[b])

        return carry

    lax.fori_loop(0, AGG_CHUNKS // _IDEPTH, body, 0)
    plsc.subcore_barrier()
    pltpu.sync_copy(acc_sh.at[pl.ds(s * 640, 640)],
                    agg_hbm.at[c, pl.ds(s * 640, 640)])


def _tc_dense_body(x_ref, th_ref, wt_ref, bt_ref, wh_ref, deg_ref,
                   y2_ref, gate_ref, gh_ref):
    xb = x_ref[...]
    d = deg_ref[:, 0] + deg_ref[:, 1]
    norm = lax.rsqrt(d + 1e-6)[:, None]
    y = jnp.dot(xb, th_ref[...], preferred_element_type=jnp.float32) * norm
    y2_ref[0] = y[:, :HH]
    y2_ref[1] = y[:, HH:]
    gate = jax.nn.sigmoid(
        jnp.dot(xb, wt_ref[...], preferred_element_type=jnp.float32)
        + bt_ref[0])
    gate_ref[...] = gate
    gh_ref[...] = (1.0 - gate) * jnp.dot(
        xb, wh_ref[...], preferred_element_type=jnp.float32)


_R = 2000


def _tc_dense(x, theta, W_t, b_t, W_h, degT):
    grid = (N // _R,)
    return pl.pallas_call(
        _tc_dense_body,
        grid=grid,
        in_specs=[
            pl.BlockSpec((_R, H), lambda i: (i, 0)),
            pl.BlockSpec((H, H), lambda i: (0, 0)),
            pl.BlockSpec((H, H), lambda i: (0, 0)),
            pl.BlockSpec((1, H), lambda i: (0, 0)),
            pl.BlockSpec((H, H), lambda i: (0, 0)),
            pl.BlockSpec((_R, NC), lambda i: (i, 0)),
        ],
        out_specs=[
            pl.BlockSpec((NC, _R, HH), lambda i: (0, i, 0)),
            pl.BlockSpec((_R, H), lambda i: (i, 0)),
            pl.BlockSpec((_R, H), lambda i: (i, 0)),
        ],
        out_shape=[
            jax.ShapeDtypeStruct((NC, N, HH), jnp.float32),
            jax.ShapeDtypeStruct((N, H), jnp.float32),
            jax.ShapeDtypeStruct((N, H), jnp.float32),
        ],
    )(x, theta, W_t, b_t, W_h, degT)


def _tc_final_body(agg_ref, gate_ref, gh_ref, deg_ref, out_ref):
    agg = jnp.concatenate([agg_ref[0], agg_ref[1]], axis=1)
    d = deg_ref[:, 0] + deg_ref[:, 1]
    norm = lax.rsqrt(d + 1e-6)[:, None]
    gate = gate_ref[...]
    z = gate * (agg * norm) + gh_ref[...]
    out_ref[...] = jnp.where(z > 0, z, jnp.exp(jnp.minimum(z, 0.0)) - 1.0)


def _tc_final(agg2, gate, gh, degT):
    grid = (N // _R,)
    return pl.pallas_call(
        _tc_final_body,
        grid=grid,
        in_specs=[
            pl.BlockSpec((NC, _R, HH), lambda i: (0, i, 0)),
            pl.BlockSpec((_R, H), lambda i: (i, 0)),
            pl.BlockSpec((_R, H), lambda i: (i, 0)),
            pl.BlockSpec((_R, NC), lambda i: (i, 0)),
        ],
        out_specs=pl.BlockSpec((_R, H), lambda i: (i, 0)),
        out_shape=jax.ShapeDtypeStruct((N, H), jnp.float32),
    )(agg2, gate, gh, degT)


def kernel(x, edge_index, W_t, b_t, W_h, theta):
    ei = edge_index.astype(jnp.int32)
    dst = ei[0]
    src = ei[1]
    dst_d = dst.reshape(NC * NS, DEG_CHUNKS, CHUNK)
    deg0, deg1 = _sc_degrees(dst_d)
    degT = jnp.stack([deg0[:N], deg1[:N]], axis=1)
    y2, gate, gh = _tc_dense(x, theta, W_t, b_t.reshape(1, H), W_h, degT)
    pad = E_PAD - E
    src_r = jnp.concatenate(
        [src, jnp.zeros((pad,), jnp.int32)]).reshape(NS, AGG_CHUNKS, ACHUNK)
    dst_r = jnp.concatenate(
        [dst, jnp.full((pad,), N, jnp.int32)]).reshape(
            NS, AGG_CHUNKS, ACHUNK)
    agg2 = _sc_aggregate(y2, src_r, dst_r)
    return _tc_final(agg2, gate, gh, degT)


# speedup vs baseline: 3.9732x; 1.6183x over previous
import functools

import jax
import jax.numpy as jnp
from jax import lax
from jax.experimental import pallas as pl
from jax.experimental.pallas import tpu as pltpu
from jax.experimental.pallas import tpu_sc as plsc

N = 10000
N_PAD = 10240
E = 160000
H = 256
HH = 128
NC = 2
NS = 16
CHUNK = 125

DEG_CHUNKS = E // (NC * NS * CHUNK)

ACHUNK = 96
AGG_CHUNKS = 108
E_TILE = AGG_CHUNKS * ACHUNK
E_PAD = NS * E_TILE
_IDEPTH = 4
_NBUF = 3

_mesh = plsc.VectorSubcoreMesh(core_axis_name="c", subcore_axis_name="s")


def _fill_const_1d(ref, n16, value):
    v = jnp.full((16,), value, dtype=jnp.float32)
    for k in range(n16):
        ref[pl.ds(k * 16, 16)] = v


@functools.partial(
    pl.kernel,
    out_type=[jax.ShapeDtypeStruct((N_PAD,), jnp.float32),
              jax.ShapeDtypeStruct((N_PAD,), jnp.float32)],
    mesh=_mesh,
    scratch_types=[
        pltpu.VMEM((DEG_CHUNKS, CHUNK), jnp.int32),
        pltpu.VMEM((128,), jnp.float32),
        pltpu.VMEM((640,), jnp.float32),
        pltpu.VMEM_SHARED((N_PAD,), jnp.float32),
    ],
)
def _sc_degrees(dst_hbm, deg0_hbm, deg1_hbm, idx_v, ones_v, zline_v, acc_sh):
    c = lax.axis_index("c")
    s = lax.axis_index("s")
    wid = c * NS + s
    pltpu.sync_copy(dst_hbm.at[wid], idx_v)
    _fill_const_1d(ones_v, 8, 1.0)
    _fill_const_1d(zline_v, 40, 0.0)
    pltpu.sync_copy(zline_v, acc_sh.at[pl.ds(s * 640, 640)])
    plsc.subcore_barrier()

    def body(j, carry):
        pltpu.sync_copy(ones_v.at[pl.ds(0, CHUNK)], acc_sh.at[idx_v.at[j]],
                        add=True)
        return carry

    lax.fori_loop(0, DEG_CHUNKS, body, 0)
    plsc.subcore_barrier()

    @pl.when(c == 0)
    def _():
        pltpu.sync_copy(acc_sh.at[pl.ds(s * 640, 640)],
                        deg0_hbm.at[pl.ds(s * 640, 640)])

    @pl.when(c == 1)
    def _():
        pltpu.sync_copy(acc_sh.at[pl.ds(s * 640, 640)],
                        deg1_hbm.at[pl.ds(s * 640, 640)])


@functools.partial(
    pl.kernel,
    out_type=jax.ShapeDtypeStruct((NC, N_PAD, HH), jnp.float32),
    mesh=_mesh,
    scratch_types=[
        pltpu.VMEM((_IDEPTH, ACHUNK), jnp.int32),
        pltpu.VMEM((_IDEPTH, ACHUNK), jnp.int32),
    ] + [pltpu.VMEM((ACHUNK, HH), jnp.float32) for _ in range(_NBUF)]
    + [pltpu.VMEM_SHARED((N_PAD, HH), jnp.float32)]
    + [pltpu.SemaphoreType.DMA for _ in range(2 * _IDEPTH + _NBUF)],
)
def _sc_aggregate(y_hbm, src_hbm, dst_hbm, agg_hbm, srci, dsti, *rest):
    bufs = rest[:_NBUF]
    acc_sh = rest[_NBUF]
    sem_si = rest[_NBUF + 1:_NBUF + 1 + _IDEPTH]
    sem_di = rest[_NBUF + 1 + _IDEPTH:_NBUF + 1 + 2 * _IDEPTH]
    sem_g = rest[_NBUF + 1 + 2 * _IDEPTH:]
    c = lax.axis_index("c")
    s = lax.axis_index("s")
    tbl = y_hbm.at[c]

    def idx_fetch(j, q):
        pltpu.async_copy(src_hbm.at[s, j], srci.at[q], sem_si[q])
        pltpu.async_copy(dst_hbm.at[s, j], dsti.at[q], sem_di[q])

    for q in range(_IDEPTH):
        idx_fetch(q, q)

    def zrow(r, carry):
        z = jnp.zeros((16,), jnp.float32)
        for k in range(HH // 16):
            bufs[0][r, pl.ds(k * 16, 16)] = z
        return carry

    lax.fori_loop(0, ACHUNK, zrow, 0)
    for t in range(6):
        pltpu.sync_copy(bufs[0], acc_sh.at[pl.ds(s * 640 + t * 96, 96)])
    pltpu.sync_copy(bufs[0].at[pl.ds(0, 64)],
                    acc_sh.at[pl.ds(s * 640 + 576, 64)])
    plsc.subcore_barrier()

    for b in range(2):
        pltpu.make_async_copy(src_hbm.at[s, b], srci.at[b], sem_si[b]).wait()
        pltpu.async_copy(tbl.at[srci.at[b]], bufs[b], sem_g[b])

    def body(jj, carry):
        j0 = jj * 12
        for u in range(12):
            b = u % _NBUF
            q = u % _IDEPTH
            pltpu.make_async_copy(tbl.at[srci.at[q]], bufs[b],
                                  sem_g[b]).wait()
            pltpu.make_async_copy(dst_hbm.at[s, j0 + u], dsti.at[q],
                                  sem_di[q]).wait()
            pltpu.sync_copy(bufs[b], acc_sh.at[dsti.at[q]], add=True)

            @pl.when(j0 + u + _IDEPTH < AGG_CHUNKS)
            def _():
                idx_fetch(j0 + u + _IDEPTH, q)

            @pl.when(j0 + u + 2 < AGG_CHUNKS)
            def _():
                q2 = (u + 2) % _IDEPTH
                b2 = (u + 2) % _NBUF
                pltpu.make_async_copy(src_hbm.at[s, j0 + u + 2],
                                      srci.at[q2], sem_si[q2]).wait()
                pltpu.async_copy(tbl.at[srci.at[q2]], bufs[b2], sem_g[b2])

        return carry

    lax.fori_loop(0, AGG_CHUNKS // 12, body, 0)
    plsc.subcore_barrier()
    pltpu.sync_copy(acc_sh.at[pl.ds(s * 640, 640)],
                    agg_hbm.at[c, pl.ds(s * 640, 640)])


def _tc_dense_body(x_ref, th_ref, wt_ref, bt_ref, wh_ref, deg_ref,
                   y2_ref, gate_ref, gh_ref):
    xb = x_ref[...]
    d = deg_ref[:, 0] + deg_ref[:, 1]
    norm = lax.rsqrt(d + 1e-6)[:, None]
    y = jnp.dot(xb, th_ref[...], preferred_element_type=jnp.float32) * norm
    y2_ref[0] = y[:, :HH]
    y2_ref[1] = y[:, HH:]
    gate = jax.nn.sigmoid(
        jnp.dot(xb, wt_ref[...], preferred_element_type=jnp.float32)
        + bt_ref[0])
    gate_ref[...] = gate
    gh_ref[...] = (1.0 - gate) * jnp.dot(
        xb, wh_ref[...], preferred_element_type=jnp.float32)


_R = 2000


def _tc_dense(x, theta, W_t, b_t, W_h, degT):
    grid = (N // _R,)
    return pl.pallas_call(
        _tc_dense_body,
        grid=grid,
        in_specs=[
            pl.BlockSpec((_R, H), lambda i: (i, 0)),
            pl.BlockSpec((H, H), lambda i: (0, 0)),
            pl.BlockSpec((H, H), lambda i: (0, 0)),
            pl.BlockSpec((1, H), lambda i: (0, 0)),
            pl.BlockSpec((H, H), lambda i: (0, 0)),
            pl.BlockSpec((_R, NC), lambda i: (i, 0)),
        ],
        out_specs=[
            pl.BlockSpec((NC, _R, HH), lambda i: (0, i, 0)),
            pl.BlockSpec((_R, H), lambda i: (i, 0)),
            pl.BlockSpec((_R, H), lambda i: (i, 0)),
        ],
        out_shape=[
            jax.ShapeDtypeStruct((NC, N, HH), jnp.float32),
            jax.ShapeDtypeStruct((N, H), jnp.float32),
            jax.ShapeDtypeStruct((N, H), jnp.float32),
        ],
    )(x, theta, W_t, b_t, W_h, degT)


def _tc_final_body(agg_ref, gate_ref, gh_ref, deg_ref, out_ref):
    agg = jnp.concatenate([agg_ref[0], agg_ref[1]], axis=1)
    d = deg_ref[:, 0] + deg_ref[:, 1]
    norm = lax.rsqrt(d + 1e-6)[:, None]
    gate = gate_ref[...]
    z = gate * (agg * norm) + gh_ref[...]
    out_ref[...] = jnp.where(z > 0, z, jnp.exp(jnp.minimum(z, 0.0)) - 1.0)


def _tc_final(agg2, gate, gh, degT):
    grid = (N // _R,)
    return pl.pallas_call(
        _tc_final_body,
        grid=grid,
        in_specs=[
            pl.BlockSpec((NC, _R, HH), lambda i: (0, i, 0)),
            pl.BlockSpec((_R, H), lambda i: (i, 0)),
            pl.BlockSpec((_R, H), lambda i: (i, 0)),
            pl.BlockSpec((_R, NC), lambda i: (i, 0)),
        ],
        out_specs=pl.BlockSpec((_R, H), lambda i: (i, 0)),
        out_shape=jax.ShapeDtypeStruct((N, H), jnp.float32),
    )(agg2, gate, gh, degT)


def kernel(x, edge_index, W_t, b_t, W_h, theta):
    ei = edge_index.astype(jnp.int32)
    dst = ei[0]
    src = ei[1]
    dst_d = dst.reshape(NC * NS, DEG_CHUNKS, CHUNK)
    deg0, deg1 = _sc_degrees(dst_d)
    degT = jnp.stack([deg0[:N], deg1[:N]], axis=1)
    y2, gate, gh = _tc_dense(x, theta, W_t, b_t.reshape(1, H), W_h, degT)
    pad = E_PAD - E
    src_r = jnp.concatenate(
        [src, jnp.zeros((pad,), jnp.int32)]).reshape(NS, AGG_CHUNKS, ACHUNK)
    dst_r = jnp.concatenate(
        [dst, jnp.full((pad,), N, jnp.int32)]).reshape(
            NS, AGG_CHUNKS, ACHUNK)
    agg2 = _sc_aggregate(y2, src_r, dst_r)
    return _tc_final(agg2, gate, gh, degT)


# speedup vs baseline: 5.0500x; 1.2710x over previous
import functools

import jax
import jax.numpy as jnp
from jax import lax
from jax.experimental import pallas as pl
from jax.experimental.pallas import tpu as pltpu
from jax.experimental.pallas import tpu_sc as plsc

N = 10000
N_PAD = 10240
E = 160000
H = 256
HH = 128
NC = 2
NS = 16
CHUNK = 125

DEG_CHUNKS = E // (NC * NS * CHUNK)

ACHUNK = 80
AGG_CHUNKS = 128
E_TILE = AGG_CHUNKS * ACHUNK
E_PAD = NS * E_TILE
_IDEPTH = 4
_NBUF = 4
_UNROLL = 4

_mesh = plsc.VectorSubcoreMesh(core_axis_name="c", subcore_axis_name="s")


def _fill_const_1d(ref, n16, value):
    v = jnp.full((16,), value, dtype=jnp.float32)
    for k in range(n16):
        ref[pl.ds(k * 16, 16)] = v


@functools.partial(
    pl.kernel,
    out_type=[jax.ShapeDtypeStruct((N_PAD,), jnp.float32),
              jax.ShapeDtypeStruct((N_PAD,), jnp.float32)],
    mesh=_mesh,
    scratch_types=[
        pltpu.VMEM((4, CHUNK), jnp.int32),
        pltpu.VMEM((128,), jnp.float32),
        pltpu.VMEM((640,), jnp.float32),
        pltpu.VMEM_SHARED((N_PAD,), jnp.float32),
    ] + [pltpu.SemaphoreType.DMA for _ in range(4)],
)
def _sc_degrees(dst_hbm, deg0_hbm, deg1_hbm, idx_v, ones_v, zline_v, acc_sh,
                *dsems):
    c = lax.axis_index("c")
    s = lax.axis_index("s")
    wid = c * NS + s
    for q in range(4):
        pltpu.async_copy(dst_hbm.at[wid, q], idx_v.at[q], dsems[q])
    _fill_const_1d(ones_v, 8, 1.0)
    _fill_const_1d(zline_v, 40, 0.0)
    pltpu.sync_copy(zline_v, acc_sh.at[pl.ds(s * 640, 640)])
    plsc.subcore_barrier()

    def body(jj, carry):
        j0 = jj * 4
        for q in range(4):
            pltpu.make_async_copy(dst_hbm.at[wid, j0 + q], idx_v.at[q],
                                  dsems[q]).wait()
            pltpu.sync_copy(ones_v.at[pl.ds(0, CHUNK)],
                            acc_sh.at[idx_v.at[q]], add=True)

            @pl.when(j0 + q + 4 < DEG_CHUNKS)
            def _():
                pltpu.async_copy(dst_hbm.at[wid, j0 + q + 4], idx_v.at[q],
                                 dsems[q])

        return carry

    lax.fori_loop(0, DEG_CHUNKS // 4, body, 0)
    plsc.subcore_barrier()

    @pl.when(c == 0)
    def _():
        pltpu.sync_copy(acc_sh.at[pl.ds(s * 640, 640)],
                        deg0_hbm.at[pl.ds(s * 640, 640)])

    @pl.when(c == 1)
    def _():
        pltpu.sync_copy(acc_sh.at[pl.ds(s * 640, 640)],
                        deg1_hbm.at[pl.ds(s * 640, 640)])


@functools.partial(
    pl.kernel,
    out_type=jax.ShapeDtypeStruct((NC, N_PAD, HH), jnp.float32),
    mesh=_mesh,
    scratch_types=[
        pltpu.VMEM((_IDEPTH, ACHUNK), jnp.int32),
        pltpu.VMEM((_IDEPTH, ACHUNK), jnp.int32),
    ] + [pltpu.VMEM((ACHUNK, HH), jnp.float32) for _ in range(_NBUF)]
    + [pltpu.VMEM_SHARED((N_PAD, HH), jnp.float32)]
    + [pltpu.SemaphoreType.DMA for _ in range(2 * _IDEPTH + _NBUF)],
)
def _sc_aggregate(y_hbm, src_hbm, dst_hbm, agg_hbm, srci, dsti, *rest):
    bufs = rest[:_NBUF]
    acc_sh = rest[_NBUF]
    sem_si = rest[_NBUF + 1:_NBUF + 1 + _IDEPTH]
    sem_di = rest[_NBUF + 1 + _IDEPTH:_NBUF + 1 + 2 * _IDEPTH]
    sem_g = rest[_NBUF + 1 + 2 * _IDEPTH:]
    c = lax.axis_index("c")
    s = lax.axis_index("s")
    tbl = y_hbm.at[c]

    def idx_fetch(j, q):
        pltpu.async_copy(src_hbm.at[s, j], srci.at[q], sem_si[q])
        pltpu.async_copy(dst_hbm.at[s, j], dsti.at[q], sem_di[q])

    for q in range(_IDEPTH):
        idx_fetch(q, q)

    def zrow(r, carry):
        z = jnp.zeros((16,), jnp.float32)
        for k in range(HH // 16):
            bufs[0][r, pl.ds(k * 16, 16)] = z
        return carry

    lax.fori_loop(0, ACHUNK, zrow, 0)
    for t in range(8):
        pltpu.sync_copy(bufs[0], acc_sh.at[pl.ds(s * 640 + t * 80, 80)])
    plsc.subcore_barrier()

    for b in range(2):
        pltpu.make_async_copy(src_hbm.at[s, b], srci.at[b], sem_si[b]).wait()
        pltpu.async_copy(tbl.at[srci.at[b]], bufs[b], sem_g[b])

    def body(jj, carry):
        j0 = jj * _UNROLL
        for u in range(_UNROLL):
            b = u % _NBUF
            q = u % _IDEPTH
            pltpu.make_async_copy(tbl.at[srci.at[q]], bufs[b],
                                  sem_g[b]).wait()
            pltpu.make_async_copy(dst_hbm.at[s, j0 + u], dsti.at[q],
                                  sem_di[q]).wait()
            pltpu.sync_copy(bufs[b], acc_sh.at[dsti.at[q]], add=True)

            @pl.when(j0 + u + _IDEPTH < AGG_CHUNKS)
            def _():
                idx_fetch(j0 + u + _IDEPTH, q)

            @pl.when(j0 + u + 2 < AGG_CHUNKS)
            def _():
                q2 = (u + 2) % _IDEPTH
                b2 = (u + 2) % _NBUF
                pltpu.make_async_copy(src_hbm.at[s, j0 + u + 2],
                                      srci.at[q2], sem_si[q2]).wait()
                pltpu.async_copy(tbl.at[srci.at[q2]], bufs[b2], sem_g[b2])

        return carry

    lax.fori_loop(0, AGG_CHUNKS // _UNROLL, body, 0)
    plsc.subcore_barrier()
    pltpu.sync_copy(acc_sh.at[pl.ds(s * 640, 640)],
                    agg_hbm.at[c, pl.ds(s * 640, 640)])


def _tc_dense_body(x_ref, th_ref, wt_ref, bt_ref, wh_ref, deg_ref,
                   y2_ref, gate_ref, gh_ref):
    xb = x_ref[...]
    d = deg_ref[:, 0] + deg_ref[:, 1]
    norm = lax.rsqrt(d + 1e-6)[:, None]
    y = jnp.dot(xb, th_ref[...], preferred_element_type=jnp.float32) * norm
    y2_ref[0] = y[:, :HH]
    y2_ref[1] = y[:, HH:]
    gate = jax.nn.sigmoid(
        jnp.dot(xb, wt_ref[...], preferred_element_type=jnp.float32)
        + bt_ref[0])
    gate_ref[...] = gate
    gh_ref[...] = (1.0 - gate) * jnp.dot(
        xb, wh_ref[...], preferred_element_type=jnp.float32)


_R = 2000


def _tc_dense(x, theta, W_t, b_t, W_h, degT):
    grid = (N // _R,)
    return pl.pallas_call(
        _tc_dense_body,
        grid=grid,
        in_specs=[
            pl.BlockSpec((_R, H), lambda i: (i, 0)),
            pl.BlockSpec((H, H), lambda i: (0, 0)),
            pl.BlockSpec((H, H), lambda i: (0, 0)),
            pl.BlockSpec((1, H), lambda i: (0, 0)),
            pl.BlockSpec((H, H), lambda i: (0, 0)),
            pl.BlockSpec((_R, NC), lambda i: (i, 0)),
        ],
        out_specs=[
            pl.BlockSpec((NC, _R, HH), lambda i: (0, i, 0)),
            pl.BlockSpec((_R, H), lambda i: (i, 0)),
            pl.BlockSpec((_R, H), lambda i: (i, 0)),
        ],
        out_shape=[
            jax.ShapeDtypeStruct((NC, N, HH), jnp.float32),
            jax.ShapeDtypeStruct((N, H), jnp.float32),
            jax.ShapeDtypeStruct((N, H), jnp.float32),
        ],
    )(x, theta, W_t, b_t, W_h, degT)


def _tc_final_body(agg_ref, gate_ref, gh_ref, deg_ref, out_ref):
    agg = jnp.concatenate([agg_ref[0], agg_ref[1]], axis=1)
    d = deg_ref[:, 0] + deg_ref[:, 1]
    norm = lax.rsqrt(d + 1e-6)[:, None]
    gate = gate_ref[...]
    z = gate * (agg * norm) + gh_ref[...]
    out_ref[...] = jnp.where(z > 0, z, jnp.exp(jnp.minimum(z, 0.0)) - 1.0)


def _tc_final(agg2, gate, gh, degT):
    grid = (N // _R,)
    return pl.pallas_call(
        _tc_final_body,
        grid=grid,
        in_specs=[
            pl.BlockSpec((NC, _R, HH), lambda i: (0, i, 0)),
            pl.BlockSpec((_R, H), lambda i: (i, 0)),
            pl.BlockSpec((_R, H), lambda i: (i, 0)),
            pl.BlockSpec((_R, NC), lambda i: (i, 0)),
        ],
        out_specs=pl.BlockSpec((_R, H), lambda i: (i, 0)),
        out_shape=jax.ShapeDtypeStruct((N, H), jnp.float32),
    )(agg2, gate, gh, degT)


def kernel(x, edge_index, W_t, b_t, W_h, theta):
    ei = edge_index.astype(jnp.int32)
    dst = ei[0]
    src = ei[1]
    dst_d = dst.reshape(NC * NS, DEG_CHUNKS, CHUNK)
    deg0, deg1 = _sc_degrees(dst_d)
    degT = jnp.stack([deg0[:N], deg1[:N]], axis=1)
    y2, gate, gh = _tc_dense(x, theta, W_t, b_t.reshape(1, H), W_h, degT)
    pad = E_PAD - E
    src_r = jnp.concatenate(
        [src, jnp.zeros((pad,), jnp.int32)]).reshape(NS, AGG_CHUNKS, ACHUNK)
    dst_r = jnp.concatenate(
        [dst, jnp.full((pad,), N, jnp.int32)]).reshape(
            NS, AGG_CHUNKS, ACHUNK)
    agg2 = _sc_aggregate(y2, src_r, dst_r)
    return _tc_final(agg2, gate, gh, degT)


# speedup vs baseline: 5.2641x; 1.0424x over previous
import functools

import jax
import jax.numpy as jnp
from jax import lax
from jax.experimental import pallas as pl
from jax.experimental.pallas import tpu as pltpu
from jax.experimental.pallas import tpu_sc as plsc

N = 10000
N_PAD = 10240
E = 160000
H = 256
HH = 128
NC = 2
NS = 16
CHUNK = 125

DEG_CHUNKS = E // (NC * NS * CHUNK)

ACHUNK = 80
AGG_CHUNKS = 128
E_TILE = AGG_CHUNKS * ACHUNK
E_PAD = NS * E_TILE
_IDEPTH = 4
_NBUF = 4
_UNROLL = 4

_mesh = plsc.VectorSubcoreMesh(core_axis_name="c", subcore_axis_name="s")


def _fill_const_1d(ref, n16, value):
    v = jnp.full((16,), value, dtype=jnp.float32)
    for k in range(n16):
        ref[pl.ds(k * 16, 16)] = v


@functools.partial(
    pl.kernel,
    out_type=[jax.ShapeDtypeStruct((N_PAD,), jnp.float32),
              jax.ShapeDtypeStruct((N_PAD,), jnp.float32)],
    mesh=_mesh,
    scratch_types=[
        pltpu.VMEM((4, CHUNK), jnp.int32),
        pltpu.VMEM((128,), jnp.float32),
        pltpu.VMEM((640,), jnp.float32),
        pltpu.VMEM_SHARED((N_PAD,), jnp.float32),
    ] + [pltpu.SemaphoreType.DMA for _ in range(4)],
)
def _sc_degrees(dst_hbm, deg0_hbm, deg1_hbm, idx_v, ones_v, zline_v, acc_sh,
                *dsems):
    c = lax.axis_index("c")
    s = lax.axis_index("s")
    wid = c * NS + s
    for q in range(4):
        pltpu.async_copy(dst_hbm.at[wid, q], idx_v.at[q], dsems[q])
    _fill_const_1d(ones_v, 8, 1.0)
    _fill_const_1d(zline_v, 40, 0.0)
    pltpu.sync_copy(zline_v, acc_sh.at[pl.ds(s * 640, 640)])
    plsc.subcore_barrier()

    def body(jj, carry):
        j0 = jj * 4
        for q in range(4):
            pltpu.make_async_copy(dst_hbm.at[wid, j0 + q], idx_v.at[q],
                                  dsems[q]).wait()
            pltpu.sync_copy(ones_v.at[pl.ds(0, CHUNK)],
                            acc_sh.at[idx_v.at[q]], add=True)

            @pl.when(j0 + q + 4 < DEG_CHUNKS)
            def _():
                pltpu.async_copy(dst_hbm.at[wid, j0 + q + 4], idx_v.at[q],
                                 dsems[q])

        return carry

    lax.fori_loop(0, DEG_CHUNKS // 4, body, 0)
    plsc.subcore_barrier()

    @pl.when(c == 0)
    def _():
        pltpu.sync_copy(acc_sh.at[pl.ds(s * 640, 640)],
                        deg0_hbm.at[pl.ds(s * 640, 640)])

    @pl.when(c == 1)
    def _():
        pltpu.sync_copy(acc_sh.at[pl.ds(s * 640, 640)],
                        deg1_hbm.at[pl.ds(s * 640, 640)])


@functools.partial(
    pl.kernel,
    out_type=jax.ShapeDtypeStruct((NC, N_PAD, HH), jnp.float32),
    mesh=_mesh,
    scratch_types=[
        pltpu.VMEM((_IDEPTH, ACHUNK), jnp.int32),
        pltpu.VMEM((_IDEPTH, ACHUNK), jnp.int32),
    ] + [pltpu.VMEM((ACHUNK, HH), jnp.float32) for _ in range(_NBUF)]
    + [pltpu.VMEM_SHARED((N_PAD, HH), jnp.float32)]
    + [pltpu.SemaphoreType.DMA for _ in range(2 * _IDEPTH + _NBUF)],
)
def _sc_aggregate(y_hbm, src_hbm, dst_hbm, agg_hbm, srci, dsti, *rest):
    bufs = rest[:_NBUF]
    acc_sh = rest[_NBUF]
    sem_si = rest[_NBUF + 1:_NBUF + 1 + _IDEPTH]
    sem_di = rest[_NBUF + 1 + _IDEPTH:_NBUF + 1 + 2 * _IDEPTH]
    sem_g = rest[_NBUF + 1 + 2 * _IDEPTH:]
    c = lax.axis_index("c")
    s = lax.axis_index("s")
    tbl = y_hbm.at[c]

    def idx_fetch(j, q):
        pltpu.async_copy(src_hbm.at[s, j], srci.at[q], sem_si[q])
        pltpu.async_copy(dst_hbm.at[s, j], dsti.at[q], sem_di[q])

    for q in range(_IDEPTH):
        idx_fetch(q, q)

    def zrow(r, carry):
        z = jnp.zeros((16,), jnp.float32)
        for k in range(HH // 16):
            bufs[0][r, pl.ds(k * 16, 16)] = z
        return carry

    lax.fori_loop(0, ACHUNK, zrow, 0)
    for t in range(8):
        pltpu.sync_copy(bufs[0], acc_sh.at[pl.ds(s * 640 + t * 80, 80)])
    plsc.subcore_barrier()

    for b in range(3):
        pltpu.make_async_copy(src_hbm.at[s, b], srci.at[b], sem_si[b]).wait()
        pltpu.async_copy(tbl.at[srci.at[b]], bufs[b], sem_g[b])

    def body(jj, carry):
        j0 = jj * _UNROLL
        for u in range(_UNROLL):
            b = u % _NBUF
            q = u % _IDEPTH
            pltpu.make_async_copy(tbl.at[srci.at[q]], bufs[b],
                                  sem_g[b]).wait()
            pltpu.make_async_copy(dst_hbm.at[s, j0 + u], dsti.at[q],
                                  sem_di[q]).wait()
            pltpu.sync_copy(bufs[b], acc_sh.at[dsti.at[q]], add=True)

            @pl.when(j0 + u + _IDEPTH < AGG_CHUNKS)
            def _():
                idx_fetch(j0 + u + _IDEPTH, q)

            @pl.when(j0 + u + 3 < AGG_CHUNKS)
            def _():
                q2 = (u + 3) % _IDEPTH
                b2 = (u + 3) % _NBUF
                pltpu.make_async_copy(src_hbm.at[s, j0 + u + 3],
                                      srci.at[q2], sem_si[q2]).wait()
                pltpu.async_copy(tbl.at[srci.at[q2]], bufs[b2], sem_g[b2])

        return carry

    lax.fori_loop(0, AGG_CHUNKS // _UNROLL, body, 0)
    plsc.subcore_barrier()
    pltpu.sync_copy(acc_sh.at[pl.ds(s * 640, 640)],
                    agg_hbm.at[c, pl.ds(s * 640, 640)])


def _tc_dense_body(x_ref, th_ref, wt_ref, bt_ref, wh_ref, deg_ref,
                   y2_ref, gate_ref, gh_ref):
    xb = x_ref[...]
    d = deg_ref[:, 0] + deg_ref[:, 1]
    norm = lax.rsqrt(d + 1e-6)[:, None]
    y = jnp.dot(xb, th_ref[...], preferred_element_type=jnp.float32) * norm
    y2_ref[0] = y[:, :HH]
    y2_ref[1] = y[:, HH:]
    gate = jax.nn.sigmoid(
        jnp.dot(xb, wt_ref[...], preferred_element_type=jnp.float32)
        + bt_ref[0])
    gate_ref[...] = gate
    gh_ref[...] = (1.0 - gate) * jnp.dot(
        xb, wh_ref[...], preferred_element_type=jnp.float32)


_R = 2000


def _tc_dense(x, theta, W_t, b_t, W_h, degT):
    grid = (N // _R,)
    return pl.pallas_call(
        _tc_dense_body,
        grid=grid,
        in_specs=[
            pl.BlockSpec((_R, H), lambda i: (i, 0)),
            pl.BlockSpec((H, H), lambda i: (0, 0)),
            pl.BlockSpec((H, H), lambda i: (0, 0)),
            pl.BlockSpec((1, H), lambda i: (0, 0)),
            pl.BlockSpec((H, H), lambda i: (0, 0)),
            pl.BlockSpec((_R, NC), lambda i: (i, 0)),
        ],
        out_specs=[
            pl.BlockSpec((NC, _R, HH), lambda i: (0, i, 0)),
            pl.BlockSpec((_R, H), lambda i: (i, 0)),
            pl.BlockSpec((_R, H), lambda i: (i, 0)),
        ],
        out_shape=[
            jax.ShapeDtypeStruct((NC, N, HH), jnp.float32),
            jax.ShapeDtypeStruct((N, H), jnp.float32),
            jax.ShapeDtypeStruct((N, H), jnp.float32),
        ],
    )(x, theta, W_t, b_t, W_h, degT)


def _tc_final_body(agg_ref, gate_ref, gh_ref, deg_ref, out_ref):
    agg = jnp.concatenate([agg_ref[0], agg_ref[1]], axis=1)
    d = deg_ref[:, 0] + deg_ref[:, 1]
    norm = lax.rsqrt(d + 1e-6)[:, None]
    gate = gate_ref[...]
    z = gate * (agg * norm) + gh_ref[...]
    out_ref[...] = jnp.where(z > 0, z, jnp.exp(jnp.minimum(z, 0.0)) - 1.0)


def _tc_final(agg2, gate, gh, degT):
    grid = (N // _R,)
    return pl.pallas_call(
        _tc_final_body,
        grid=grid,
        in_specs=[
            pl.BlockSpec((NC, _R, HH), lambda i: (0, i, 0)),
            pl.BlockSpec((_R, H), lambda i: (i, 0)),
            pl.BlockSpec((_R, H), lambda i: (i, 0)),
            pl.BlockSpec((_R, NC), lambda i: (i, 0)),
        ],
        out_specs=pl.BlockSpec((_R, H), lambda i: (i, 0)),
        out_shape=jax.ShapeDtypeStruct((N, H), jnp.float32),
    )(agg2, gate, gh, degT)


def kernel(x, edge_index, W_t, b_t, W_h, theta):
    ei = edge_index.astype(jnp.int32)
    dst = ei[0]
    src = ei[1]
    dst_d = dst.reshape(NC * NS, DEG_CHUNKS, CHUNK)
    deg0, deg1 = _sc_degrees(dst_d)
    degT = jnp.stack([deg0[:N], deg1[:N]], axis=1)
    y2, gate, gh = _tc_dense(x, theta, W_t, b_t.reshape(1, H), W_h, degT)
    pad = E_PAD - E
    src_r = jnp.concatenate(
        [src, jnp.zeros((pad,), jnp.int32)]).reshape(NS, AGG_CHUNKS, ACHUNK)
    dst_r = jnp.concatenate(
        [dst, jnp.full((pad,), N, jnp.int32)]).reshape(
            NS, AGG_CHUNKS, ACHUNK)
    agg2 = _sc_aggregate(y2, src_r, dst_r)
    return _tc_final(agg2, gate, gh, degT)
